# baseline (speedup 1.0000x reference)
import dataclasses
import functools

import jax
import jax.numpy as jnp
from jax import lax
from jax.experimental import pallas as pl
from jax.experimental.pallas import tpu as pltpu
from jax.experimental.pallas import tpu_sc as plsc

_SIZE = 4096
_LANES = 16
_NUM_CORES = 2
_NUM_SUBCORES = 16
_S_BLK = 256


def _compiler_params():
    cp = pltpu.CompilerParams()
    if "needs_layout_passes" in pltpu.CompilerParams.__dataclass_fields__:
        cp = dataclasses.replace(cp, needs_layout_passes=False)
    return cp


def kernel(sorted_lmbd, N_array_areas, areas, weight):
    B, S, Lp1 = sorted_lmbd.shape
    L = Lp1 - 1

    lmbd_t = jnp.transpose(sorted_lmbd, (2, 0, 1))
    n_t = jnp.transpose(N_array_areas, (0, 2, 1))
    a_t = jnp.transpose(areas, (0, 2, 1))

    mesh = plsc.VectorSubcoreMesh(
        core_axis_name="c", subcore_axis_name="s",
        num_cores=_NUM_CORES, num_subcores=_NUM_SUBCORES,
    )

    @functools.partial(
        pl.kernel,
        out_type=(
            jax.ShapeDtypeStruct((B, S), jnp.float32),
            jax.ShapeDtypeStruct((_SIZE,), jnp.float32),
        ),
        mesh=mesh,
        scratch_types=[pltpu.VMEM((_SIZE,), jnp.float32)],
        compiler_params=_compiler_params(),
    )
    def sc_kernel(lmbd_hbm, n_hbm, a_hbm, w_hbm, score_hbm, wabs_hbm, table_v):
        pltpu.sync_copy(w_hbm, table_v)

        @pl.loop(0, _SIZE, step=_LANES)
        def _(i):
            sl = pl.ds(i, _LANES)
            table_v[sl] = jnp.abs(table_v[sl])

        wid = lax.axis_index("s") * _NUM_CORES + lax.axis_index("c")

        @pl.when(wid == 0)
        def _():
            pltpu.sync_copy(table_v, wabs_hbm)

        def body(lmbd_vm, n_vm, a_vm, out_vm):
            @plsc.parallel_loop(0, _S_BLK, step=_LANES, unroll=4)
            def _(s0):
                sl = pl.ds(s0, _LANES)
                acc = None
                for l in range(L):
                    lm = lmbd_vm[l + 1, 0, sl]
                    lmf = jnp.minimum(lm * float(_SIZE), float(_SIZE - 1))
                    seg = plsc.load_gather(table_v, [lmf.astype(jnp.int32)])
                    term = seg * n_vm[0, l, sl] * a_vm[0, l, sl]
                    acc = term if acc is None else acc + term
                out_vm[0, sl] = jnp.clip(acc, 0.0, 2.0)

        n_sb = S // _S_BLK
        pltpu.emit_pipeline(
            body,
            grid=(B * n_sb,),
            in_specs=[
                pl.BlockSpec((Lp1, 1, _S_BLK),
                             lambda i: (0, i // n_sb, i % n_sb)),
                pl.BlockSpec((1, L, _S_BLK),
                             lambda i: (i // n_sb, 0, i % n_sb)),
                pl.BlockSpec((1, L, _S_BLK),
                             lambda i: (i // n_sb, 0, i % n_sb)),
            ],
            out_specs=[pl.BlockSpec((1, _S_BLK),
                                    lambda i: (i // n_sb, i % n_sb))],
            core_axis_name=("c", "s"),
            dimension_semantics=(pltpu.PARALLEL,),
        )(lmbd_hbm, n_hbm, a_hbm, score_hbm)

    score, wabs = sc_kernel(lmbd_t, n_t, a_t, weight)
    return score, wabs


# speedup vs baseline: 1.4488x; 1.4488x over previous
import dataclasses
import functools

import jax
import jax.numpy as jnp
from jax import lax
from jax.experimental import pallas as pl
from jax.experimental.pallas import tpu as pltpu
from jax.experimental.pallas import tpu_sc as plsc

_SIZE = 4096
_LANES = 16
_NUM_CORES = 2
_NUM_SUBCORES = 16
_S_BLK = 256


def _compiler_params():
    cp = pltpu.CompilerParams()
    if "needs_layout_passes" in pltpu.CompilerParams.__dataclass_fields__:
        cp = dataclasses.replace(cp, needs_layout_passes=False)
    return cp


def kernel(sorted_lmbd, N_array_areas, areas, weight):
    B, S, Lp1 = sorted_lmbd.shape
    L = Lp1 - 1

    lmbd_t = jnp.transpose(sorted_lmbd, (2, 0, 1))
    n_t = jnp.transpose(N_array_areas, (0, 2, 1))
    a_t = jnp.transpose(areas, (0, 2, 1))

    mesh = plsc.VectorSubcoreMesh(
        core_axis_name="c", subcore_axis_name="s",
        num_cores=_NUM_CORES, num_subcores=_NUM_SUBCORES,
    )

    @functools.partial(
        pl.kernel,
        out_type=(
            jax.ShapeDtypeStruct((B, S), jnp.float32),
            jax.ShapeDtypeStruct((_SIZE,), jnp.float32),
        ),
        mesh=mesh,
        scratch_types=[pltpu.VMEM((_SIZE,), jnp.float32)],
        compiler_params=_compiler_params(),
    )
    def sc_kernel(lmbd_hbm, n_hbm, a_hbm, w_hbm, score_hbm, wabs_hbm, table_v):
        pltpu.sync_copy(w_hbm, table_v)

        @pl.loop(0, _SIZE, step=_LANES)
        def _(i):
            sl = pl.ds(i, _LANES)
            table_v[sl] = jnp.abs(table_v[sl])

        wid = lax.axis_index("s") * _NUM_CORES + lax.axis_index("c")

        @pl.when(wid == 0)
        def _():
            pltpu.sync_copy(table_v, wabs_hbm)

        def body(lmbd_vm, n_vm, a_vm, out_vm):
            @plsc.parallel_loop(0, _S_BLK, step=_LANES, unroll=2)
            def _(s0):
                sl = pl.ds(s0, _LANES)
                acc = None
                for l in range(L):
                    lm = lmbd_vm[l + 1, 0, sl]
                    lmf = jnp.minimum(lm * float(_SIZE), float(_SIZE - 1))
                    seg = plsc.load_gather(table_v, [lmf.astype(jnp.int32)])
                    term = seg * n_vm[0, l, sl] * a_vm[0, l, sl]
                    acc = term if acc is None else acc + term
                out_vm[0, sl] = jnp.clip(acc, 0.0, 2.0)

        n_sb = S // _S_BLK
        pltpu.emit_pipeline(
            body,
            grid=(B * n_sb,),
            in_specs=[
                pl.BlockSpec((Lp1, 1, _S_BLK),
                             lambda i: (0, i // n_sb, i % n_sb)),
                pl.BlockSpec((1, L, _S_BLK),
                             lambda i: (i // n_sb, 0, i % n_sb)),
                pl.BlockSpec((1, L, _S_BLK),
                             lambda i: (i // n_sb, 0, i % n_sb)),
            ],
            out_specs=[pl.BlockSpec((1, _S_BLK),
                                    lambda i: (i // n_sb, i % n_sb))],
            core_axis_name=("c", "s"),
            dimension_semantics=(pltpu.PARALLEL,),
        )(lmbd_hbm, n_hbm, a_hbm, score_hbm)

    score, wabs = sc_kernel(lmbd_t, n_t, a_t, weight)
    return score, wabs
